# baseline (speedup 1.0000x reference)
import functools

import jax
import jax.numpy as jnp
from jax import lax
from jax.experimental import pallas as pl
from jax.experimental.pallas import tpu as pltpu
from jax.experimental.pallas import tpu_sc as plsc

BATCH = 16384
EMB = 16
NC = 2
NS = 16
NW = NC * NS
BPW = BATCH // NW
L = 16
CH = 128
NCHUNK = BPW // CH
NFULL = 1000000 // 128
PAIRS = 122
NROW128 = 1000000 * EMB // 128


def _mesh():
    return plsc.VectorSubcoreMesh(core_axis_name="c", subcore_axis_name="s")


def _transpose_block(blk, obuf, rowi):
    for g in range(8):
        vs = [plsc.load_gather(blk, [rowi, jnp.full((L,), g * 16 + t, jnp.int32)])
              for t in range(16)]
        for t in range(16):
            e = g * 16 + t
            obuf[e // 8, pl.ds((e % 8) * L, L)] = vs[t]


@functools.partial(
    pl.kernel,
    out_type=(
        jax.ShapeDtypeStruct((NROW128, 128), jnp.float32),
        jax.ShapeDtypeStruct((NROW128, 128), jnp.float32),
    ),
    mesh=_mesh(),
    compiler_params=pltpu.CompilerParams(needs_layout_passes=False),
    scratch_types=[
        pltpu.VMEM((16, 128), jnp.float32),
        pltpu.VMEM((16, 128), jnp.float32),
        pltpu.VMEM((16, 128), jnp.float32),
        pltpu.VMEM((16, 128), jnp.float32),
        pltpu.VMEM((16, 128), jnp.float32),
        pltpu.VMEM((16, 128), jnp.float32),
        pltpu.VMEM((16, 128), jnp.float32),
        pltpu.VMEM((16, 128), jnp.float32),
        pltpu.SemaphoreType.DMA,
        pltpu.SemaphoreType.DMA,
        pltpu.SemaphoreType.DMA,
        pltpu.SemaphoreType.DMA,
    ],
)
def _relayout(uet_hbm, met_hbm, tailu_hbm, tailm_hbm, uout_hbm, mout_hbm,
              ublk0, ublk1, mblk0, mblk1, uob0, uob1, mob0, mob1,
              semui, semmi, semuo, semmo):
    wid = lax.axis_index("s") * NC + lax.axis_index("c")
    lo = wid * (2 * PAIRS)
    rowi = lax.iota(jnp.int32, L)

    def fire_in(b, ub, mb):
        off = pl.multiple_of(b * 128, 128)
        pltpu.async_copy(uet_hbm.at[:, pl.ds(off, 128)], ub, semui)
        pltpu.async_copy(met_hbm.at[:, pl.ds(off, 128)], mb, semmi)

    def fire_out(b, uo, mo):
        r = pl.multiple_of(b * 16, 16)
        pltpu.async_copy(uo, uout_hbm.at[pl.ds(r, 16), :], semuo)
        pltpu.async_copy(mo, mout_hbm.at[pl.ds(r, 16), :], semmo)

    def wait_in():
        pltpu.make_async_copy(uet_hbm.at[:, pl.ds(0, 128)], ublk0, semui).wait()
        pltpu.make_async_copy(met_hbm.at[:, pl.ds(0, 128)], mblk0, semmi).wait()

    def wait_out():
        pltpu.make_async_copy(uob0, uout_hbm.at[pl.ds(0, 16), :], semuo).wait()
        pltpu.make_async_copy(mob0, mout_hbm.at[pl.ds(0, 16), :], semmo).wait()

    fire_in(lo, ublk0, mblk0)

    def pair(i, _):
        b0 = lo + 2 * i
        fire_in(b0 + 1, ublk1, mblk1)
        wait_in()

        @pl.when(i >= 1)
        def _():
            wait_out()
            wait_out()
        _transpose_block(ublk0, uob0, rowi)
        _transpose_block(mblk0, mob0, rowi)
        fire_out(b0, uob0, mob0)

        @pl.when(i < PAIRS - 1)
        def _():
            fire_in(b0 + 2, ublk0, mblk0)
        wait_in()
        _transpose_block(ublk1, uob1, rowi)
        _transpose_block(mblk1, mob1, rowi)
        fire_out(b0 + 1, uob1, mob1)
        return 0

    lax.fori_loop(0, PAIRS, pair, 0)
    wait_out()
    wait_out()

    @pl.when(wid < NFULL - 32 * 2 * PAIRS)
    def _():
        b = 32 * 2 * PAIRS + wid
        off = pl.multiple_of(b * 128, 128)
        pltpu.sync_copy(uet_hbm.at[:, pl.ds(off, 128)], ublk0)
        pltpu.sync_copy(met_hbm.at[:, pl.ds(off, 128)], mblk0)
        _transpose_block(ublk0, uob0, rowi)
        _transpose_block(mblk0, mob0, rowi)
        pltpu.sync_copy(uob0, uout_hbm.at[pl.ds(b * 16, 16), :])
        pltpu.sync_copy(mob0, mout_hbm.at[pl.ds(b * 16, 16), :])

    @pl.when(wid == 4)
    def _():
        pltpu.sync_copy(tailu_hbm, ublk0.at[pl.ds(0, 8), :])
        pltpu.sync_copy(ublk0.at[pl.ds(0, 8), :],
                        uout_hbm.at[pl.ds(NROW128 - 8, 8), :])

    @pl.when(wid == 5)
    def _():
        pltpu.sync_copy(tailm_hbm, mblk0.at[pl.ds(0, 8), :])
        pltpu.sync_copy(mblk0.at[pl.ds(0, 8), :],
                        mout_hbm.at[pl.ds(NROW128 - 8, 8), :])


@functools.partial(
    pl.kernel,
    out_type=jax.ShapeDtypeStruct((NW * EMB,), jnp.float32),
    mesh=_mesh(),
    compiler_params=pltpu.CompilerParams(needs_layout_passes=False),
    scratch_types=[
        pltpu.VMEM((BPW,), jnp.int32),
        pltpu.VMEM((BPW,), jnp.int32),
        pltpu.VMEM((BPW,), jnp.int32),
        pltpu.VMEM((BPW,), jnp.int32),
        pltpu.VMEM((CH, 128), jnp.float32),
        pltpu.VMEM((CH, 128), jnp.float32),
        pltpu.VMEM((CH, 128), jnp.float32),
        pltpu.VMEM((CH, 128), jnp.float32),
        pltpu.VMEM((EMB,), jnp.float32),
        pltpu.SemaphoreType.DMA,
        pltpu.SemaphoreType.DMA,
        pltpu.SemaphoreType.DMA,
        pltpu.SemaphoreType.DMA,
    ],
)
def _gather_partials(uidx_hbm, midx_hbm, ue_hbm, me_hbm, part_hbm,
                     urow_v, mrow_v, uoff_v, moff_v,
                     ubuf0, ubuf1, mbuf0, mbuf1, acc_v,
                     semu0, semu1, semm0, semm1):
    wid = lax.axis_index("s") * NC + lax.axis_index("c")
    base = wid * BPW
    pltpu.sync_copy(uidx_hbm.at[pl.ds(base, BPW)], urow_v)
    pltpu.sync_copy(midx_hbm.at[pl.ds(base, BPW)], mrow_v)
    for i in range(BPW // L):
        sl = pl.ds(i * L, L)
        u = urow_v[sl]
        m = mrow_v[sl]
        uoff_v[sl] = lax.shift_left(jnp.bitwise_and(u, 7), 4)
        moff_v[sl] = lax.shift_left(jnp.bitwise_and(m, 7), 4)
        urow_v[sl] = lax.shift_right_logical(u, 3)
        mrow_v[sl] = lax.shift_right_logical(m, 3)

    ubufs = (ubuf0, ubuf1)
    mbufs = (mbuf0, mbuf1)
    usems = (semu0, semu1)
    msems = (semm0, semm1)

    def fire(c):
        sl = pl.ds(c * CH, CH)
        cu = pltpu.async_copy(ue_hbm.at[urow_v.at[sl]], ubufs[c % 2], usems[c % 2])
        cm = pltpu.async_copy(me_hbm.at[mrow_v.at[sl]], mbufs[c % 2], msems[c % 2])
        return cu, cm

    inflight = fire(0)
    acc = jnp.zeros((L,), jnp.float32)
    rowi = lax.iota(jnp.int32, L)
    for c in range(NCHUNK):
        cu, cm = inflight
        if c + 1 < NCHUNK:
            nxt = fire(c + 1)
        cu.wait()
        cm.wait()
        ub = ubufs[c % 2]
        mb = mbufs[c % 2]

        def gbody(g, acc, _c=c, _ub=ub, _mb=mb):
            ri = rowi + g * L
            uo = plsc.load_gather(uoff_v, [ri + _c * CH])
            mo = plsc.load_gather(moff_v, [ri + _c * CH])
            for j in range(L):
                uj = plsc.load_gather(_ub, [ri, uo + j])
                mj = plsc.load_gather(_mb, [ri, mo + j])
                acc = acc + uj * mj
            return acc

        acc = lax.fori_loop(0, CH // L, gbody, acc)
        if c + 1 < NCHUNK:
            inflight = nxt
    acc_v[...] = acc
    pltpu.sync_copy(acc_v, part_hbm.at[pl.ds(wid * EMB, EMB)])


@functools.partial(
    pl.kernel,
    out_type=jax.ShapeDtypeStruct((BATCH,), jnp.float32),
    mesh=_mesh(),
    compiler_params=pltpu.CompilerParams(use_tc_tiling_on_sc=False,
                                         needs_layout_passes=False),
    scratch_types=[
        pltpu.VMEM((NW * EMB,), jnp.float32),
        pltpu.VMEM((BPW,), jnp.int32),
        pltpu.VMEM((BPW,), jnp.int32),
        pltpu.VMEM((BPW,), jnp.float32),
        pltpu.VMEM((BPW,), jnp.float32),
        pltpu.VMEM((BPW,), jnp.float32),
        pltpu.SemaphoreType.DMA,
        pltpu.SemaphoreType.DMA,
        pltpu.SemaphoreType.DMA,
    ],
)
def _reduce_sigmoid(part_hbm, uidx_hbm, midx_hbm, ub_hbm, mb_hbm, out_hbm,
                    part_v, uidx_v, midx_v, ub_v, mb_v, out_v,
                    sem0, sem1, sem2):
    wid = lax.axis_index("s") * NC + lax.axis_index("c")
    base = wid * BPW
    cp = pltpu.async_copy(part_hbm, part_v, sem2)
    pltpu.sync_copy(uidx_hbm.at[pl.ds(base, BPW)], uidx_v)
    pltpu.sync_copy(midx_hbm.at[pl.ds(base, BPW)], midx_v)
    c0 = pltpu.async_copy(ub_hbm.at[uidx_v], ub_v, sem0)
    c1 = pltpu.async_copy(mb_hbm.at[midx_v], mb_v, sem1)
    cp.wait()
    acc = part_v[pl.ds(0, L)]
    for j in range(1, NW):
        acc = acc + part_v[pl.ds(j * EMB, L)]
    total = lax.reduce_sum_p.bind(acc, axes=(0,))
    c0.wait()
    c1.wait()
    for i in range(BPW // L):
        sl = pl.ds(i * L, L)
        x = ub_v[sl] + mb_v[sl] + total
        out_v[sl] = 1.0 / (1.0 + jnp.exp(-x))
    pltpu.sync_copy(out_v, out_hbm.at[pl.ds(base, BPW)])


def kernel(inputs, user_embedding, user_bias, movie_embedding, movie_bias):
    uidx = inputs[:, 0]
    midx = inputs[:, 1]
    uet = user_embedding.T
    met = movie_embedding.T
    tailu = user_embedding[1000000 - 64:, :].reshape(8, 128)
    tailm = movie_embedding[1000000 - 64:, :].reshape(8, 128)
    u128, m128 = _relayout(uet, met, tailu, tailm)
    part = _gather_partials(uidx, midx, u128, m128)
    out = _reduce_sigmoid(part, uidx, midx, user_bias.reshape(-1),
                          movie_bias.reshape(-1))
    return out.reshape(BATCH, 1)


# speedup vs baseline: 1.0096x; 1.0096x over previous
import functools

import jax
import jax.numpy as jnp
from jax import lax
from jax.experimental import pallas as pl
from jax.experimental.pallas import tpu as pltpu
from jax.experimental.pallas import tpu_sc as plsc

BATCH = 16384
EMB = 16
NC = 2
NS = 16
NW = NC * NS
BPW = BATCH // NW
L = 16
CH = 128
NCHUNK = BPW // CH
NFULL = 1000000 // 128
PAIRS = 122
NROW128 = 1000000 * EMB // 128


def _mesh():
    return plsc.VectorSubcoreMesh(core_axis_name="c", subcore_axis_name="s")


def _transpose_block(blk, obuf, diag):
    rowi, hs, scols, srow0s = diag
    for k in range(8):
        vs = [plsc.load_gather(blk, [rowi, hs[d] + (16 * k)]) for d in range(L)]
        for d in range(L):
            plsc.store_scatter(obuf, [srow0s[d] + (2 * k), scols[d]], vs[d])


@functools.partial(
    pl.kernel,
    out_type=(
        jax.ShapeDtypeStruct((NROW128, 128), jnp.float32),
        jax.ShapeDtypeStruct((NROW128, 128), jnp.float32),
    ),
    mesh=_mesh(),
    compiler_params=pltpu.CompilerParams(needs_layout_passes=False),
    scratch_types=[
        pltpu.VMEM((16, 128), jnp.float32),
        pltpu.VMEM((16, 128), jnp.float32),
        pltpu.VMEM((16, 128), jnp.float32),
        pltpu.VMEM((16, 128), jnp.float32),
        pltpu.VMEM((16, 128), jnp.float32),
        pltpu.VMEM((16, 128), jnp.float32),
        pltpu.VMEM((16, 128), jnp.float32),
        pltpu.VMEM((16, 128), jnp.float32),
        pltpu.SemaphoreType.DMA,
        pltpu.SemaphoreType.DMA,
        pltpu.SemaphoreType.DMA,
        pltpu.SemaphoreType.DMA,
    ],
)
def _relayout(uet_hbm, met_hbm, tailu_hbm, tailm_hbm, uout_hbm, mout_hbm,
              ublk0, ublk1, mblk0, mblk1, uob0, uob1, mob0, mob1,
              semui, semmi, semuo, semmo):
    wid = lax.axis_index("s") * NC + lax.axis_index("c")
    lo = wid * (2 * PAIRS)
    rowi = lax.iota(jnp.int32, L)
    hs = [jnp.bitwise_and(rowi + d, 15) for d in range(L)]
    g16s = [lax.shift_left(h, 4) + rowi for h in hs]
    scols = [jnp.bitwise_and(g, 127) for g in g16s]
    srow0s = [lax.shift_right_logical(g, 7) for g in g16s]
    diag = (rowi, hs, scols, srow0s)

    def fire_in(b, ub, mb):
        off = pl.multiple_of(b * 128, 128)
        pltpu.async_copy(uet_hbm.at[:, pl.ds(off, 128)], ub, semui)
        pltpu.async_copy(met_hbm.at[:, pl.ds(off, 128)], mb, semmi)

    def fire_out(b, uo, mo):
        r = pl.multiple_of(b * 16, 16)
        pltpu.async_copy(uo, uout_hbm.at[pl.ds(r, 16), :], semuo)
        pltpu.async_copy(mo, mout_hbm.at[pl.ds(r, 16), :], semmo)

    def wait_in():
        pltpu.make_async_copy(uet_hbm.at[:, pl.ds(0, 128)], ublk0, semui).wait()
        pltpu.make_async_copy(met_hbm.at[:, pl.ds(0, 128)], mblk0, semmi).wait()

    def wait_out():
        pltpu.make_async_copy(uob0, uout_hbm.at[pl.ds(0, 16), :], semuo).wait()
        pltpu.make_async_copy(mob0, mout_hbm.at[pl.ds(0, 16), :], semmo).wait()

    fire_in(lo, ublk0, mblk0)

    def pair(i, _):
        b0 = lo + 2 * i
        fire_in(b0 + 1, ublk1, mblk1)
        wait_in()

        @pl.when(i >= 1)
        def _():
            wait_out()
            wait_out()
        _transpose_block(ublk0, uob0, diag)
        _transpose_block(mblk0, mob0, diag)
        fire_out(b0, uob0, mob0)

        @pl.when(i < PAIRS - 1)
        def _():
            fire_in(b0 + 2, ublk0, mblk0)
        wait_in()
        _transpose_block(ublk1, uob1, diag)
        _transpose_block(mblk1, mob1, diag)
        fire_out(b0 + 1, uob1, mob1)
        return 0

    lax.fori_loop(0, PAIRS, pair, 0)
    wait_out()
    wait_out()

    @pl.when(wid < NFULL - 32 * 2 * PAIRS)
    def _():
        b = 32 * 2 * PAIRS + wid
        off = pl.multiple_of(b * 128, 128)
        pltpu.sync_copy(uet_hbm.at[:, pl.ds(off, 128)], ublk0)
        pltpu.sync_copy(met_hbm.at[:, pl.ds(off, 128)], mblk0)
        _transpose_block(ublk0, uob0, diag)
        _transpose_block(mblk0, mob0, diag)
        pltpu.sync_copy(uob0, uout_hbm.at[pl.ds(b * 16, 16), :])
        pltpu.sync_copy(mob0, mout_hbm.at[pl.ds(b * 16, 16), :])

    @pl.when(wid == 4)
    def _():
        pltpu.sync_copy(tailu_hbm, ublk0.at[pl.ds(0, 8), :])
        pltpu.sync_copy(ublk0.at[pl.ds(0, 8), :],
                        uout_hbm.at[pl.ds(NROW128 - 8, 8), :])

    @pl.when(wid == 5)
    def _():
        pltpu.sync_copy(tailm_hbm, mblk0.at[pl.ds(0, 8), :])
        pltpu.sync_copy(mblk0.at[pl.ds(0, 8), :],
                        mout_hbm.at[pl.ds(NROW128 - 8, 8), :])


@functools.partial(
    pl.kernel,
    out_type=jax.ShapeDtypeStruct((NW * EMB,), jnp.float32),
    mesh=_mesh(),
    compiler_params=pltpu.CompilerParams(needs_layout_passes=False),
    scratch_types=[
        pltpu.VMEM((BPW,), jnp.int32),
        pltpu.VMEM((BPW,), jnp.int32),
        pltpu.VMEM((BPW,), jnp.int32),
        pltpu.VMEM((BPW,), jnp.int32),
        pltpu.VMEM((CH, 128), jnp.float32),
        pltpu.VMEM((CH, 128), jnp.float32),
        pltpu.VMEM((CH, 128), jnp.float32),
        pltpu.VMEM((CH, 128), jnp.float32),
        pltpu.VMEM((EMB,), jnp.float32),
        pltpu.SemaphoreType.DMA,
        pltpu.SemaphoreType.DMA,
        pltpu.SemaphoreType.DMA,
        pltpu.SemaphoreType.DMA,
    ],
)
def _gather_partials(uidx_hbm, midx_hbm, ue_hbm, me_hbm, part_hbm,
                     urow_v, mrow_v, uoff_v, moff_v,
                     ubuf0, ubuf1, mbuf0, mbuf1, acc_v,
                     semu0, semu1, semm0, semm1):
    wid = lax.axis_index("s") * NC + lax.axis_index("c")
    base = wid * BPW
    pltpu.sync_copy(uidx_hbm.at[pl.ds(base, BPW)], urow_v)
    pltpu.sync_copy(midx_hbm.at[pl.ds(base, BPW)], mrow_v)
    for i in range(BPW // L):
        sl = pl.ds(i * L, L)
        u = urow_v[sl]
        m = mrow_v[sl]
        uoff_v[sl] = lax.shift_left(jnp.bitwise_and(u, 7), 4)
        moff_v[sl] = lax.shift_left(jnp.bitwise_and(m, 7), 4)
        urow_v[sl] = lax.shift_right_logical(u, 3)
        mrow_v[sl] = lax.shift_right_logical(m, 3)

    ubufs = (ubuf0, ubuf1)
    mbufs = (mbuf0, mbuf1)
    usems = (semu0, semu1)
    msems = (semm0, semm1)

    def fire(c):
        sl = pl.ds(c * CH, CH)
        cu = pltpu.async_copy(ue_hbm.at[urow_v.at[sl]], ubufs[c % 2], usems[c % 2])
        cm = pltpu.async_copy(me_hbm.at[mrow_v.at[sl]], mbufs[c % 2], msems[c % 2])
        return cu, cm

    inflight = fire(0)
    acc = jnp.zeros((L,), jnp.float32)
    rowi = lax.iota(jnp.int32, L)
    for c in range(NCHUNK):
        cu, cm = inflight
        if c + 1 < NCHUNK:
            nxt = fire(c + 1)
        cu.wait()
        cm.wait()
        ub = ubufs[c % 2]
        mb = mbufs[c % 2]

        def gbody(g, acc, _c=c, _ub=ub, _mb=mb):
            ri = rowi + g * L
            uo = plsc.load_gather(uoff_v, [ri + _c * CH])
            mo = plsc.load_gather(moff_v, [ri + _c * CH])
            for j in range(L):
                uj = plsc.load_gather(_ub, [ri, uo + j])
                mj = plsc.load_gather(_mb, [ri, mo + j])
                acc = acc + uj * mj
            return acc

        acc = lax.fori_loop(0, CH // L, gbody, acc)
        if c + 1 < NCHUNK:
            inflight = nxt
    acc_v[...] = acc
    pltpu.sync_copy(acc_v, part_hbm.at[pl.ds(wid * EMB, EMB)])


@functools.partial(
    pl.kernel,
    out_type=jax.ShapeDtypeStruct((BATCH,), jnp.float32),
    mesh=_mesh(),
    compiler_params=pltpu.CompilerParams(use_tc_tiling_on_sc=False,
                                         needs_layout_passes=False),
    scratch_types=[
        pltpu.VMEM((NW * EMB,), jnp.float32),
        pltpu.VMEM((BPW,), jnp.int32),
        pltpu.VMEM((BPW,), jnp.int32),
        pltpu.VMEM((BPW,), jnp.float32),
        pltpu.VMEM((BPW,), jnp.float32),
        pltpu.VMEM((BPW,), jnp.float32),
        pltpu.SemaphoreType.DMA,
        pltpu.SemaphoreType.DMA,
        pltpu.SemaphoreType.DMA,
    ],
)
def _reduce_sigmoid(part_hbm, uidx_hbm, midx_hbm, ub_hbm, mb_hbm, out_hbm,
                    part_v, uidx_v, midx_v, ub_v, mb_v, out_v,
                    sem0, sem1, sem2):
    wid = lax.axis_index("s") * NC + lax.axis_index("c")
    base = wid * BPW
    cp = pltpu.async_copy(part_hbm, part_v, sem2)
    pltpu.sync_copy(uidx_hbm.at[pl.ds(base, BPW)], uidx_v)
    pltpu.sync_copy(midx_hbm.at[pl.ds(base, BPW)], midx_v)
    c0 = pltpu.async_copy(ub_hbm.at[uidx_v], ub_v, sem0)
    c1 = pltpu.async_copy(mb_hbm.at[midx_v], mb_v, sem1)
    cp.wait()
    acc = part_v[pl.ds(0, L)]
    for j in range(1, NW):
        acc = acc + part_v[pl.ds(j * EMB, L)]
    total = lax.reduce_sum_p.bind(acc, axes=(0,))
    c0.wait()
    c1.wait()
    for i in range(BPW // L):
        sl = pl.ds(i * L, L)
        x = ub_v[sl] + mb_v[sl] + total
        out_v[sl] = 1.0 / (1.0 + jnp.exp(-x))
    pltpu.sync_copy(out_v, out_hbm.at[pl.ds(base, BPW)])


def kernel(inputs, user_embedding, user_bias, movie_embedding, movie_bias):
    uidx = inputs[:, 0]
    midx = inputs[:, 1]
    uet = user_embedding.T
    met = movie_embedding.T
    tailu = user_embedding[1000000 - 64:, :].reshape(8, 128)
    tailm = movie_embedding[1000000 - 64:, :].reshape(8, 128)
    u128, m128 = _relayout(uet, met, tailu, tailm)
    part = _gather_partials(uidx, midx, u128, m128)
    out = _reduce_sigmoid(part, uidx, midx, user_bias.reshape(-1),
                          movie_bias.reshape(-1))
    return out.reshape(BATCH, 1)


# speedup vs baseline: 1.0138x; 1.0042x over previous
import functools

import jax
import jax.numpy as jnp
from jax import lax
from jax.experimental import pallas as pl
from jax.experimental.pallas import tpu as pltpu
from jax.experimental.pallas import tpu_sc as plsc

BATCH = 16384
EMB = 16
NC = 2
NS = 16
NW = NC * NS
BPW = BATCH // NW
L = 16
CH = 128
NCHUNK = BPW // CH
NFULL = 1000000 // 128
PAIRS = 122
NROW128 = 1000000 * EMB // 128


def _mesh():
    return plsc.VectorSubcoreMesh(core_axis_name="c", subcore_axis_name="s")


def _transpose_block(blk, obuf, rowi):
    for g in range(16):
        vs = [plsc.load_gather(blk, [rowi, jnp.full((L,), g * 8 + t, jnp.int32)])
              for t in range(8)]
        for t in range(8):
            obuf[g, pl.ds(t * L, L)] = vs[t]


@functools.partial(
    pl.kernel,
    out_type=(
        jax.ShapeDtypeStruct((NROW128, 128), jnp.float32),
        jax.ShapeDtypeStruct((NROW128, 128), jnp.float32),
    ),
    mesh=_mesh(),
    compiler_params=pltpu.CompilerParams(needs_layout_passes=False),
    scratch_types=[
        pltpu.VMEM((16, 128), jnp.float32),
        pltpu.VMEM((16, 128), jnp.float32),
        pltpu.VMEM((16, 128), jnp.float32),
        pltpu.VMEM((16, 128), jnp.float32),
        pltpu.VMEM((16, 128), jnp.float32),
        pltpu.VMEM((16, 128), jnp.float32),
        pltpu.VMEM((16, 128), jnp.float32),
        pltpu.VMEM((16, 128), jnp.float32),
        pltpu.SemaphoreType.DMA,
        pltpu.SemaphoreType.DMA,
        pltpu.SemaphoreType.DMA,
        pltpu.SemaphoreType.DMA,
    ],
)
def _relayout(uet_hbm, met_hbm, tailu_hbm, tailm_hbm, uout_hbm, mout_hbm,
              ublk0, ublk1, mblk0, mblk1, uob0, uob1, mob0, mob1,
              semui, semmi, semuo, semmo):
    wid = lax.axis_index("s") * NC + lax.axis_index("c")
    lo = wid * (2 * PAIRS)
    rowi = lax.iota(jnp.int32, L)

    def fire_in(b, ub, mb):
        off = pl.multiple_of(b * 128, 128)
        pltpu.async_copy(uet_hbm.at[:, pl.ds(off, 128)], ub, semui)
        pltpu.async_copy(met_hbm.at[:, pl.ds(off, 128)], mb, semmi)

    def fire_out(b, uo, mo):
        r = pl.multiple_of(b * 16, 16)
        pltpu.async_copy(uo, uout_hbm.at[pl.ds(r, 16), :], semuo)
        pltpu.async_copy(mo, mout_hbm.at[pl.ds(r, 16), :], semmo)

    def wait_in():
        pltpu.make_async_copy(uet_hbm.at[:, pl.ds(0, 128)], ublk0, semui).wait()
        pltpu.make_async_copy(met_hbm.at[:, pl.ds(0, 128)], mblk0, semmi).wait()

    def wait_out():
        pltpu.make_async_copy(uob0, uout_hbm.at[pl.ds(0, 16), :], semuo).wait()
        pltpu.make_async_copy(mob0, mout_hbm.at[pl.ds(0, 16), :], semmo).wait()

    fire_in(lo, ublk0, mblk0)

    def pair(i, _):
        b0 = lo + 2 * i
        fire_in(b0 + 1, ublk1, mblk1)
        wait_in()

        @pl.when(i >= 1)
        def _():
            wait_out()
            wait_out()
        _transpose_block(ublk0, uob0, rowi)
        _transpose_block(mblk0, mob0, rowi)
        fire_out(b0, uob0, mob0)

        @pl.when(i < PAIRS - 1)
        def _():
            fire_in(b0 + 2, ublk0, mblk0)
        wait_in()
        _transpose_block(ublk1, uob1, rowi)
        _transpose_block(mblk1, mob1, rowi)
        fire_out(b0 + 1, uob1, mob1)
        return 0

    lax.fori_loop(0, PAIRS, pair, 0)
    wait_out()
    wait_out()

    @pl.when(wid < NFULL - 32 * 2 * PAIRS)
    def _():
        b = 32 * 2 * PAIRS + wid
        off = pl.multiple_of(b * 128, 128)
        pltpu.sync_copy(uet_hbm.at[:, pl.ds(off, 128)], ublk0)
        pltpu.sync_copy(met_hbm.at[:, pl.ds(off, 128)], mblk0)
        _transpose_block(ublk0, uob0, rowi)
        _transpose_block(mblk0, mob0, rowi)
        pltpu.sync_copy(uob0, uout_hbm.at[pl.ds(b * 16, 16), :])
        pltpu.sync_copy(mob0, mout_hbm.at[pl.ds(b * 16, 16), :])

    @pl.when(wid == 4)
    def _():
        pltpu.sync_copy(tailu_hbm, ublk0.at[pl.ds(0, 8), :])
        pltpu.sync_copy(ublk0.at[pl.ds(0, 8), :],
                        uout_hbm.at[pl.ds(NROW128 - 8, 8), :])

    @pl.when(wid == 5)
    def _():
        pltpu.sync_copy(tailm_hbm, mblk0.at[pl.ds(0, 8), :])
        pltpu.sync_copy(mblk0.at[pl.ds(0, 8), :],
                        mout_hbm.at[pl.ds(NROW128 - 8, 8), :])


@functools.partial(
    pl.kernel,
    out_type=jax.ShapeDtypeStruct((NW * EMB,), jnp.float32),
    mesh=_mesh(),
    compiler_params=pltpu.CompilerParams(needs_layout_passes=False),
    scratch_types=[
        pltpu.VMEM((BPW,), jnp.int32),
        pltpu.VMEM((BPW,), jnp.int32),
        pltpu.VMEM((BPW,), jnp.int32),
        pltpu.VMEM((BPW,), jnp.int32),
        pltpu.VMEM((CH, 128), jnp.float32),
        pltpu.VMEM((CH, 128), jnp.float32),
        pltpu.VMEM((CH, 128), jnp.float32),
        pltpu.VMEM((CH, 128), jnp.float32),
        pltpu.VMEM((EMB,), jnp.float32),
        pltpu.SemaphoreType.DMA,
        pltpu.SemaphoreType.DMA,
        pltpu.SemaphoreType.DMA,
        pltpu.SemaphoreType.DMA,
    ],
)
def _gather_partials(uidx_hbm, midx_hbm, ue_hbm, me_hbm, part_hbm,
                     urow_v, mrow_v, uoff_v, moff_v,
                     ubuf0, ubuf1, mbuf0, mbuf1, acc_v,
                     semu0, semu1, semm0, semm1):
    wid = lax.axis_index("s") * NC + lax.axis_index("c")
    base = wid * BPW
    pltpu.sync_copy(uidx_hbm.at[pl.ds(base, BPW)], urow_v)
    pltpu.sync_copy(midx_hbm.at[pl.ds(base, BPW)], mrow_v)
    for i in range(BPW // L):
        sl = pl.ds(i * L, L)
        u = urow_v[sl]
        m = mrow_v[sl]
        uoff_v[sl] = lax.shift_left(jnp.bitwise_and(u, 7), 4)
        moff_v[sl] = lax.shift_left(jnp.bitwise_and(m, 7), 4)
        urow_v[sl] = lax.shift_right_logical(u, 3)
        mrow_v[sl] = lax.shift_right_logical(m, 3)

    ubufs = (ubuf0, ubuf1)
    mbufs = (mbuf0, mbuf1)
    usems = (semu0, semu1)
    msems = (semm0, semm1)

    def fire(c):
        sl = pl.ds(c * CH, CH)
        cu = pltpu.async_copy(ue_hbm.at[urow_v.at[sl]], ubufs[c % 2], usems[c % 2])
        cm = pltpu.async_copy(me_hbm.at[mrow_v.at[sl]], mbufs[c % 2], msems[c % 2])
        return cu, cm

    inflight = fire(0)
    acc = jnp.zeros((L,), jnp.float32)
    rowi = lax.iota(jnp.int32, L)
    for c in range(NCHUNK):
        cu, cm = inflight
        if c + 1 < NCHUNK:
            nxt = fire(c + 1)
        cu.wait()
        cm.wait()
        ub = ubufs[c % 2]
        mb = mbufs[c % 2]

        def gbody(g, acc, _c=c, _ub=ub, _mb=mb):
            ri = rowi + g * L
            uo = plsc.load_gather(uoff_v, [ri + _c * CH])
            mo = plsc.load_gather(moff_v, [ri + _c * CH])
            for j in range(L):
                uj = plsc.load_gather(_ub, [ri, uo + j])
                mj = plsc.load_gather(_mb, [ri, mo + j])
                acc = acc + uj * mj
            return acc

        acc = lax.fori_loop(0, CH // L, gbody, acc)
        if c + 1 < NCHUNK:
            inflight = nxt
    acc_v[...] = acc
    pltpu.sync_copy(acc_v, part_hbm.at[pl.ds(wid * EMB, EMB)])


@functools.partial(
    pl.kernel,
    out_type=jax.ShapeDtypeStruct((BATCH,), jnp.float32),
    mesh=_mesh(),
    compiler_params=pltpu.CompilerParams(use_tc_tiling_on_sc=False,
                                         needs_layout_passes=False),
    scratch_types=[
        pltpu.VMEM((NW * EMB,), jnp.float32),
        pltpu.VMEM((BPW,), jnp.int32),
        pltpu.VMEM((BPW,), jnp.int32),
        pltpu.VMEM((BPW,), jnp.float32),
        pltpu.VMEM((BPW,), jnp.float32),
        pltpu.VMEM((BPW,), jnp.float32),
        pltpu.SemaphoreType.DMA,
        pltpu.SemaphoreType.DMA,
        pltpu.SemaphoreType.DMA,
    ],
)
def _reduce_sigmoid(part_hbm, uidx_hbm, midx_hbm, ub_hbm, mb_hbm, out_hbm,
                    part_v, uidx_v, midx_v, ub_v, mb_v, out_v,
                    sem0, sem1, sem2):
    wid = lax.axis_index("s") * NC + lax.axis_index("c")
    base = wid * BPW
    cp = pltpu.async_copy(part_hbm, part_v, sem2)
    pltpu.sync_copy(uidx_hbm.at[pl.ds(base, BPW)], uidx_v)
    pltpu.sync_copy(midx_hbm.at[pl.ds(base, BPW)], midx_v)
    c0 = pltpu.async_copy(ub_hbm.at[uidx_v], ub_v, sem0)
    c1 = pltpu.async_copy(mb_hbm.at[midx_v], mb_v, sem1)
    cp.wait()
    acc = part_v[pl.ds(0, L)]
    for j in range(1, NW):
        acc = acc + part_v[pl.ds(j * EMB, L)]
    total = lax.reduce_sum_p.bind(acc, axes=(0,))
    c0.wait()
    c1.wait()
    for i in range(BPW // L):
        sl = pl.ds(i * L, L)
        x = ub_v[sl] + mb_v[sl] + total
        out_v[sl] = 1.0 / (1.0 + jnp.exp(-x))
    pltpu.sync_copy(out_v, out_hbm.at[pl.ds(base, BPW)])


def kernel(inputs, user_embedding, user_bias, movie_embedding, movie_bias):
    uidx = inputs[:, 0]
    midx = inputs[:, 1]
    uet = user_embedding.T
    met = movie_embedding.T
    tailu = user_embedding[1000000 - 64:, :].reshape(8, 128)
    tailm = movie_embedding[1000000 - 64:, :].reshape(8, 128)
    u128, m128 = _relayout(uet, met, tailu, tailm)
    part = _gather_partials(uidx, midx, u128, m128)
    out = _reduce_sigmoid(part, uidx, midx, user_bias.reshape(-1),
                          movie_bias.reshape(-1))
    return out.reshape(BATCH, 1)
